# Initial kernel scaffold; baseline (speedup 1.0000x reference)
import jax
import jax.numpy as jnp
from jax.experimental import pallas as pl


def kernel(batch, embed_weight, fc1_w, fc1_b):
    raise NotImplementedError("write your pallas kernel here")


# speedup vs baseline: 1.0026x; 1.0026x over previous
import functools

import jax
import jax.numpy as jnp
from jax import lax
from jax.experimental import pallas as pl
from jax.experimental.pallas import tpu as pltpu
from jax.experimental.pallas import tpu_sc as plsc

EMBED = 64
HIST = 50
NLANE = 16
NCHUNK = EMBED // NLANE

NW = 32
NBUF = 4


def _cbow_pool_body(table_hbm, batch_hbm, out_hbm, idx_v, bufs, out_v, sems,
                    *, bpw):
    wid = lax.axis_index("s") * 2 + lax.axis_index("c")
    base = wid * bpw

    pltpu.sync_copy(batch_hbm.at[pl.ds(base, bpw)], idx_v)

    def _gather(b, k):
        pltpu.make_async_copy(
            table_hbm.at[idx_v.at[b]], bufs.at[k], sems.at[k]).start()

    def _wait(b, k):
        pltpu.make_async_copy(
            table_hbm.at[idx_v.at[b]], bufs.at[k], sems.at[k]).wait()

    for k in range(NBUF):
        _gather(k, k)

    def g_body(g, carry):
        for k in range(NBUF):
            b = g * NBUF + k
            _wait(b, k)
            accs = [bufs[k, 0, pl.ds(NLANE * j, NLANE)] for j in range(NCHUNK)]
            for h in range(1, HIST):
                for j in range(NCHUNK):
                    accs[j] = accs[j] + bufs[k, h, pl.ds(NLANE * j, NLANE)]
            for j in range(NCHUNK):
                out_v[b, pl.ds(NLANE * j, NLANE)] = accs[j]
            nb = b + NBUF

            @pl.when(nb < bpw)
            def _():
                _gather(nb, k)
        return carry

    lax.fori_loop(0, bpw // NBUF, g_body, 0)

    pltpu.sync_copy(out_v, out_hbm.at[pl.ds(base, bpw)])


def _cbow_pool(table, batch):
    batch_size = batch.shape[0]
    bpw = batch_size // NW
    mesh = plsc.VectorSubcoreMesh(core_axis_name="c", subcore_axis_name="s")
    k = pl.kernel(
        functools.partial(_cbow_pool_body, bpw=bpw),
        out_type=jax.ShapeDtypeStruct((batch_size, EMBED), jnp.float32),
        mesh=mesh,
        scratch_types=[
            pltpu.VMEM((bpw, HIST), jnp.int32),
            pltpu.VMEM((NBUF, HIST, EMBED), jnp.float32),
            pltpu.VMEM((bpw, EMBED), jnp.float32),
            pltpu.SemaphoreType.DMA((NBUF,)),
        ],
        compiler_params=pltpu.CompilerParams(use_tc_tiling_on_sc=False),
    )
    return k(table, batch)


def _linear_body(x_ref, w_ref, b_ref, o_ref):
    o_ref[...] = jnp.dot(
        x_ref[...], w_ref[...], preferred_element_type=jnp.float32
    ) + b_ref[...]


def _linear(x, w, b):
    batch_size = x.shape[0]
    labels = w.shape[1]
    return pl.pallas_call(
        _linear_body,
        out_shape=jax.ShapeDtypeStruct((batch_size, labels), jnp.float32),
    )(x, w, b.reshape(1, labels))


def kernel(batch, embed_weight, fc1_w, fc1_b):
    batch = batch.astype(jnp.int32)
    cont_bow = _cbow_pool(embed_weight, batch)
    return _linear(cont_bow, fc1_w, fc1_b)
